# baseline (speedup 1.0000x reference)
import functools

import jax
import jax.numpy as jnp
from jax import lax
from jax.experimental import pallas as pl
from jax.experimental.pallas import tpu as pltpu
from jax.experimental.pallas import tpu_sc as plsc

B, N, E, D, H = 16, 2048, 32768, 128, 128
NC, NS = 2, 16
CH = 128
KC = (E // NS) // CH
DEGW = 16
GPC = B // NC
KD = (E * B // (NC * NS)) // CH
NPS = N // NS

_mesh = plsc.VectorSubcoreMesh(core_axis_name="core", subcore_axis_name="subcore")


Q = 4
DQ = D // Q
PB = 8
KB = (E // CH) // PB
KD2 = (E // NC) // CH


def _fill_rows(ref, value):
    rows, width = ref.shape

    @pl.loop(0, rows)
    def _(r):
        for l in range(width // 16):
            ref[r, pl.ds(l * 16, 16)] = jnp.full((16,), value, jnp.float32)


KDg = E // CH
PBd = 8
KBd = KDg // PBd

_DEG_SCRATCH = [
    pltpu.VMEM((CH, DEGW), jnp.float32),
    pltpu.VMEM((KD2, CH), jnp.int32),
    pltpu.VMEM((CH, DEGW), jnp.float32),
    pltpu.VMEM((N // CH, CH), jnp.int32),
    pltpu.VMEM((N, DEGW), jnp.float32),
    pltpu.VMEM_SHARED((NS, N, DEGW), jnp.float32),
]


def _build_identity(ref):
    iota = lax.iota(jnp.int32, 16)

    @pl.loop(0, ref.shape[0])
    def _(k):
        for l in range(CH // 16):
            ref[k, pl.ds(l * 16, 16)] = k * CH + l * 16 + iota


def _sc_degree_body(dst_hbm, out_hbm, ones_v, idx_v, zero_v, id_v, stage_v,
                    acc):
    c = lax.axis_index("core")
    s = lax.axis_index("subcore")
    g = s
    half = c

    _fill_rows(ones_v, 1.0)
    _fill_rows(zero_v, 0.0)
    _build_identity(id_v)
    my = acc.at[s]

    @pl.loop(0, N // CH)
    def _(k):
        pltpu.sync_copy(zero_v, my.at[id_v.at[k]])

    pltpu.sync_copy(dst_hbm.at[g, half], idx_v)

    @pl.loop(0, KD2)
    def _(k):
        pltpu.sync_copy(ones_v, my.at[idx_v.at[k]], add=True)

    pltpu.sync_copy(my, stage_v)
    pltpu.sync_copy(stage_v, out_hbm.at[g, half])


_SEG_SCRATCH = [
    pltpu.VMEM((KB, CH), jnp.int32),
    pltpu.VMEM((KB, CH), jnp.int32),
    pltpu.VMEM((6, CH, DQ), jnp.float32),
    pltpu.VMEM((CH, DQ), jnp.float32),
    pltpu.VMEM((N // CH, CH), jnp.int32),
    pltpu.SemaphoreType.DMA,
    pltpu.SemaphoreType.DMA,
    pltpu.VMEM_SHARED((NS, N, DQ), jnp.float32),
]


def _sc_segsum_body(hp_hbm, src_hbm, dst_hbm, out_hbm, src_v, dst_v,
                    rows_v, zero_v, id_v, gsem, ssem, acc):
    c = lax.axis_index("core")
    s = lax.axis_index("subcore")
    w = s * NC + c
    q = w // (B // NC)
    gs = w % (B // NC)

    _fill_rows(zero_v, 0.0)
    _build_identity(id_v)
    my = acc.at[s]

    @pl.loop(0, NC)
    def _(jj):
        g = gs + (B // NC) * jj

        @pl.loop(0, N // CH)
        def _(k):
            pltpu.sync_copy(zero_v, my.at[id_v.at[k]])

        @pl.loop(0, PB)
        def _(p):
            pltpu.sync_copy(src_hbm.at[q, g, p], src_v)
            pltpu.sync_copy(dst_hbm.at[g, p], dst_v)

            gds = [None] * 6
            for j in range(min(3, KB)):
                gds[j] = pltpu.async_copy(hp_hbm.at[src_v.at[j]],
                                          rows_v.at[j], gsem)
            sds = [None, None]
            for k in range(KB):
                b = k % 6
                gds[b].wait()
                if sds[k & 1] is not None:
                    sds[k & 1].wait()
                if k + 3 < KB:
                    b2 = (k + 3) % 6
                    gds[b2] = pltpu.async_copy(hp_hbm.at[src_v.at[k + 3]],
                                               rows_v.at[b2], gsem)
                sds[k & 1] = pltpu.async_copy(rows_v.at[b], my.at[dst_v.at[k]],
                                              ssem, add=True)
            for sd in sds:
                if sd is not None:
                    sd.wait()

        pltpu.sync_copy(my, out_hbm.at[q, g])


_SC_PARAMS = pltpu.CompilerParams(use_tc_tiling_on_sc=False)

_sc_degree = pl.kernel(
    _sc_degree_body,
    out_type=jax.ShapeDtypeStruct((B, NC, N, DEGW), jnp.float32),
    mesh=_mesh,
    scratch_types=_DEG_SCRATCH,
    compiler_params=_SC_PARAMS,
)

_sc_segsum = pl.kernel(
    _sc_segsum_body,
    out_type=jax.ShapeDtypeStruct((Q, B, N, DQ), jnp.float32),
    mesh=_mesh,
    scratch_types=_SEG_SCRATCH,
    compiler_params=_SC_PARAMS,
)


def _dinv_col(deg_ref):
    p = deg_ref[0, 0] + deg_ref[0, 1]
    return lax.rsqrt(p[:, 0:1] + 1.0)


def _cat(ref):
    return jnp.concatenate([ref[q, 0] for q in range(Q)], axis=1)


def _split_out(o_ref, mat):
    for q in range(Q):
        o_ref[q, 0] = mat[:, q * DQ:(q + 1) * DQ]


def _nt(a, b):
    return lax.dot_general(a, b, (((1,), (1,)), ((), ())),
                           preferred_element_type=jnp.float32)


def _tc1_body(deg_ref, x_ref, w_ref, hp_ref):
    dinv = _dinv_col(deg_ref)
    _split_out(hp_ref, _nt(x_ref[0], w_ref[...]) * dinv)


def _tc2_body(deg_ref, s_ref, hp_ref, b_ref, w_ref, o_ref):
    dinv = _dinv_col(deg_ref)
    gmat = jnp.maximum((_cat(s_ref) + _cat(hp_ref)) * dinv
                       + b_ref[...][None, :], 0.0)
    _split_out(o_ref, _nt(gmat, w_ref[...]) * dinv)


def _tc3_body(deg_ref, s_ref, hp_ref, b_ref, x_ref, encw_ref, encb_ref,
              w1_ref, b1_ref, w2_ref, b2_ref, w3_ref, b3_ref,
              zn_ref, xi_ref):
    g = pl.program_id(0)
    dinv = _dinv_col(deg_ref)
    out2 = (_cat(s_ref) + _cat(hp_ref)) * dinv + b_ref[...][None, :]
    emb = jnp.sum(out2, axis=0, keepdims=True) * (1.0 / N)
    embn = emb / jnp.maximum(jnp.sqrt(jnp.sum(emb * emb)), 1e-12)
    h1 = jnp.maximum(_nt(embn, w1_ref[...]) + b1_ref[...][None, :], 0.0)
    h2 = jnp.maximum(_nt(h1, w2_ref[...]) + b2_ref[...][None, :], 0.0)
    xi_ref[0] = _nt(h2, w3_ref[...]) + b3_ref[...][None, :]

    @pl.when(g == 0)
    def _():
        z = _nt(x_ref[...], encw_ref[...]) + encb_ref[...][None, :]
        zn_ref[...] = z / jnp.maximum(
            jnp.sqrt(jnp.sum(z * z, axis=1, keepdims=True)), 1e-12)


def _full(shape):
    return pl.BlockSpec(shape, lambda g: tuple(0 for _ in shape))


_DEG_SPEC = pl.BlockSpec((1, NC, N, DEGW), lambda g: (g, 0, 0, 0))
_MAT_SPEC = pl.BlockSpec((1, N, D), lambda g: (g, 0, 0))
_SEG_SPEC = pl.BlockSpec((Q, 1, N, DQ), lambda g: (0, g, 0, 0))


def kernel(x, labels, loc, sub_x, sub_edge_index, enc_W, enc_b, conv1_W,
           conv1_b, conv2_W, conv2_b, imp_W1, imp_b1, imp_W2, imp_b2,
           imp_W3, imp_b3):
    src = sub_edge_index[:, 0, :].astype(jnp.int32)
    dst = sub_edge_index[:, 1, :].astype(jnp.int32)

    dst_deg = dst.reshape(B, NC, KD2, CH)
    qg_off = ((jnp.arange(Q, dtype=jnp.int32) * B * N)[:, None]
              + (jnp.arange(B, dtype=jnp.int32) * N)[None, :])
    srcq = (src[None, :, :] + qg_off[:, :, None]).reshape(Q, B, PB, KB, CH)
    dstl = dst.reshape(B, PB, KB, CH)

    degp = _sc_degree(dst_deg)

    hp1 = pl.pallas_call(
        _tc1_body,
        grid=(B,),
        in_specs=[_DEG_SPEC, _MAT_SPEC, _full((H, D))],
        out_specs=_SEG_SPEC,
        out_shape=jax.ShapeDtypeStruct((Q, B, N, DQ), jnp.float32),
    )(degp, sub_x, conv1_W)

    s1 = _sc_segsum(hp1.reshape(Q * B * N, DQ), srcq, dstl)

    hp2 = pl.pallas_call(
        _tc2_body,
        grid=(B,),
        in_specs=[_DEG_SPEC, _SEG_SPEC, _SEG_SPEC, _full((H,)), _full((H, H))],
        out_specs=_SEG_SPEC,
        out_shape=jax.ShapeDtypeStruct((Q, B, N, DQ), jnp.float32),
    )(degp, s1, hp1, conv1_b, conv2_W)

    s2 = _sc_segsum(hp2.reshape(Q * B * N, DQ), srcq, dstl)

    z_norm, x_imp = pl.pallas_call(
        _tc3_body,
        grid=(B,),
        in_specs=[_DEG_SPEC, _SEG_SPEC, _SEG_SPEC, _full((H,)),
                  _full((B, D)), _full((H, D)), _full((H,)),
                  _full((128, H)), _full((128,)),
                  _full((256, 128)), _full((256,)),
                  _full((D, 256)), _full((D,))],
        out_specs=[_full((B, D)), pl.BlockSpec((1, 1, D), lambda g: (g, 0, 0))],
        out_shape=[jax.ShapeDtypeStruct((B, D), jnp.float32),
                   jax.ShapeDtypeStruct((B, 1, D), jnp.float32)],
    )(degp, s2, hp2, conv2_b, x, enc_W, enc_b,
      imp_W1, imp_b1, imp_W2, imp_b2, imp_W3, imp_b3)

    return (z_norm, x_imp.reshape(B, D))


# speedup vs baseline: 1.1324x; 1.1324x over previous
import functools

import jax
import jax.numpy as jnp
from jax import lax
from jax.experimental import pallas as pl
from jax.experimental.pallas import tpu as pltpu
from jax.experimental.pallas import tpu_sc as plsc

B, N, E, D, H = 16, 2048, 32768, 128, 128
NC, NS = 2, 16
CH = 128
KC = (E // NS) // CH
DEGW = 16
GPC = B // NC
KD = (E * B // (NC * NS)) // CH
NPS = N // NS

_mesh = plsc.VectorSubcoreMesh(core_axis_name="core", subcore_axis_name="subcore")


Q = 4
DQ = D // Q
PB = 8
KB = (E // CH) // PB
KD2 = (E // NC) // CH


def _fill_rows(ref, value):
    rows, width = ref.shape

    @pl.loop(0, rows)
    def _(r):
        for l in range(width // 16):
            ref[r, pl.ds(l * 16, 16)] = jnp.full((16,), value, jnp.float32)


KDg = E // CH
PBd = 8
KBd = KDg // PBd

_DEG_SCRATCH = [
    pltpu.VMEM((CH, DEGW), jnp.float32),
    pltpu.VMEM((KD2, CH), jnp.int32),
    pltpu.VMEM((CH, DEGW), jnp.float32),
    pltpu.VMEM((N // CH, CH), jnp.int32),
    pltpu.VMEM((N, DEGW), jnp.float32),
    pltpu.VMEM_SHARED((NS, N, DEGW), jnp.float32),
]


def _build_identity(ref):
    iota = lax.iota(jnp.int32, 16)

    @pl.loop(0, ref.shape[0])
    def _(k):
        for l in range(CH // 16):
            ref[k, pl.ds(l * 16, 16)] = k * CH + l * 16 + iota


def _sc_degree_body(dst_hbm, out_hbm, ones_v, idx_v, zero_v, id_v, stage_v,
                    acc):
    c = lax.axis_index("core")
    s = lax.axis_index("subcore")
    g = s
    half = c

    _fill_rows(ones_v, 1.0)
    _fill_rows(zero_v, 0.0)
    _build_identity(id_v)
    my = acc.at[s]

    @pl.loop(0, N // CH)
    def _(k):
        pltpu.sync_copy(zero_v, my.at[id_v.at[k]])

    pltpu.sync_copy(dst_hbm.at[g, half], idx_v)

    @pl.loop(0, KD2)
    def _(k):
        pltpu.sync_copy(ones_v, my.at[idx_v.at[k]], add=True)

    pltpu.sync_copy(my, stage_v)
    pltpu.sync_copy(stage_v, out_hbm.at[g, half])


_SEG_SCRATCH = [
    pltpu.VMEM((KB, CH), jnp.int32),
    pltpu.VMEM((KB, CH), jnp.int32),
    pltpu.VMEM((6, CH, DQ), jnp.float32),
    pltpu.VMEM((CH, DQ), jnp.float32),
    pltpu.VMEM((N // CH, CH), jnp.int32),
    pltpu.SemaphoreType.DMA,
    pltpu.SemaphoreType.DMA,
    pltpu.VMEM_SHARED((NS, N, DQ), jnp.float32),
]


HB = B // NC


def _sc_segsum_body(hp_hbm, src_hbm, dst_hbm, out_hbm, src_v, dst_v,
                    rows_v, zero_v, id_v, gsem, ssem, acc):
    c = lax.axis_index("core")
    s = lax.axis_index("subcore")
    w = s * NC + c
    q = w // HB
    g = w % HB

    _fill_rows(zero_v, 0.0)
    _build_identity(id_v)
    my = acc.at[s]

    @pl.loop(0, N // CH)
    def _(k):
        pltpu.sync_copy(zero_v, my.at[id_v.at[k]])

    @pl.loop(0, PB)
    def _(p):
        pltpu.sync_copy(src_hbm.at[q, g, p], src_v)
        pltpu.sync_copy(dst_hbm.at[g, p], dst_v)

        gds = [None] * 6
        for j in range(min(3, KB)):
            gds[j] = pltpu.async_copy(hp_hbm.at[src_v.at[j]],
                                      rows_v.at[j], gsem)
        sds = [None, None]
        for k in range(KB):
            b = k % 6
            gds[b].wait()
            if sds[k & 1] is not None:
                sds[k & 1].wait()
            if k + 3 < KB:
                b2 = (k + 3) % 6
                gds[b2] = pltpu.async_copy(hp_hbm.at[src_v.at[k + 3]],
                                           rows_v.at[b2], gsem)
            sds[k & 1] = pltpu.async_copy(rows_v.at[b], my.at[dst_v.at[k]],
                                          ssem, add=True)
        for sd in sds:
            if sd is not None:
                sd.wait()

    pltpu.sync_copy(my, out_hbm.at[q, g])


_SC_PARAMS = pltpu.CompilerParams(use_tc_tiling_on_sc=False)

_sc_degree = pl.kernel(
    _sc_degree_body,
    out_type=jax.ShapeDtypeStruct((B, NC, N, DEGW), jnp.float32),
    mesh=_mesh,
    scratch_types=_DEG_SCRATCH,
    compiler_params=_SC_PARAMS,
)

_sc_segsum = pl.kernel(
    _sc_segsum_body,
    out_type=jax.ShapeDtypeStruct((Q, HB, N, DQ), jnp.float32),
    mesh=_mesh,
    scratch_types=_SEG_SCRATCH,
    compiler_params=_SC_PARAMS,
)


def _dinv_col(deg_ref):
    p = deg_ref[0, 0] + deg_ref[0, 1]
    return lax.rsqrt(p[:, 0:1] + 1.0)


def _cat(ref):
    return jnp.concatenate([ref[q, 0] for q in range(Q)], axis=1)


def _split_out(o_ref, mat):
    for q in range(Q):
        o_ref[q, 0] = mat[:, q * DQ:(q + 1) * DQ]


def _nt(a, b):
    return lax.dot_general(a, b, (((1,), (1,)), ((), ())),
                           preferred_element_type=jnp.float32)


def _tc1_body(deg_ref, x_ref, w_ref, hp_ref):
    dinv = _dinv_col(deg_ref)
    _split_out(hp_ref, _nt(x_ref[0], w_ref[...]) * dinv)


def _tc2_body(deg_ref, s_ref, hp_ref, b_ref, w_ref, o_ref):
    dinv = _dinv_col(deg_ref)
    gmat = jnp.maximum((_cat(s_ref) + _cat(hp_ref)) * dinv
                       + b_ref[...][None, :], 0.0)
    _split_out(o_ref, _nt(gmat, w_ref[...]) * dinv)


def _tc3_body(with_z, deg_ref, s_ref, hp_ref, b_ref, x_ref, encw_ref,
              encb_ref, w1_ref, b1_ref, w2_ref, b2_ref, w3_ref, b3_ref,
              *out_refs):
    if with_z:
        zn_ref, xi_ref = out_refs
    else:
        (xi_ref,) = out_refs
    g = pl.program_id(0)
    dinv = _dinv_col(deg_ref)
    out2 = (_cat(s_ref) + _cat(hp_ref)) * dinv + b_ref[...][None, :]
    emb = jnp.sum(out2, axis=0, keepdims=True) * (1.0 / N)
    embn = emb / jnp.maximum(jnp.sqrt(jnp.sum(emb * emb)), 1e-12)
    h1 = jnp.maximum(_nt(embn, w1_ref[...]) + b1_ref[...][None, :], 0.0)
    h2 = jnp.maximum(_nt(h1, w2_ref[...]) + b2_ref[...][None, :], 0.0)
    xi_ref[0] = _nt(h2, w3_ref[...]) + b3_ref[...][None, :]

    if with_z:
        @pl.when(g == 0)
        def _():
            z = _nt(x_ref[...], encw_ref[...]) + encb_ref[...][None, :]
            zn_ref[...] = z / jnp.maximum(
                jnp.sqrt(jnp.sum(z * z, axis=1, keepdims=True)), 1e-12)


def _full(shape):
    return pl.BlockSpec(shape, lambda g: tuple(0 for _ in shape))


_DEG_SPEC = pl.BlockSpec((1, NC, N, DEGW), lambda g: (g, 0, 0, 0))
_MAT_SPEC = pl.BlockSpec((1, N, D), lambda g: (g, 0, 0))
_SEG_SPEC = pl.BlockSpec((Q, 1, N, DQ), lambda g: (0, g, 0, 0))


def kernel(x, labels, loc, sub_x, sub_edge_index, enc_W, enc_b, conv1_W,
           conv1_b, conv2_W, conv2_b, imp_W1, imp_b1, imp_W2, imp_b2,
           imp_W3, imp_b3):
    src = sub_edge_index[:, 0, :].astype(jnp.int32)
    dst = sub_edge_index[:, 1, :].astype(jnp.int32)

    dst_deg = dst.reshape(B, NC, KD2, CH)
    degp = _sc_degree(dst_deg)

    qg_off = ((jnp.arange(Q, dtype=jnp.int32) * HB * N)[:, None]
              + (jnp.arange(HB, dtype=jnp.int32) * N)[None, :])

    def half_chain(lo, with_z):
        srch = src[lo:lo + HB]
        srcq = (srch[None] + qg_off[:, :, None]).reshape(Q, HB, PB, KB, CH)
        dstl = dst[lo:lo + HB].reshape(HB, PB, KB, CH)
        degh = degp[lo:lo + HB]

        hp1 = pl.pallas_call(
            _tc1_body,
            grid=(HB,),
            in_specs=[_DEG_SPEC, _MAT_SPEC, _full((H, D))],
            out_specs=_SEG_SPEC,
            out_shape=jax.ShapeDtypeStruct((Q, HB, N, DQ), jnp.float32),
        )(degh, sub_x[lo:lo + HB], conv1_W)

        s1 = _sc_segsum(hp1.reshape(Q * HB * N, DQ), srcq, dstl)

        hp2 = pl.pallas_call(
            _tc2_body,
            grid=(HB,),
            in_specs=[_DEG_SPEC, _SEG_SPEC, _SEG_SPEC, _full((H,)),
                      _full((H, H))],
            out_specs=_SEG_SPEC,
            out_shape=jax.ShapeDtypeStruct((Q, HB, N, DQ), jnp.float32),
        )(degh, s1, hp1, conv1_b, conv2_W)

        s2 = _sc_segsum(hp2.reshape(Q * HB * N, DQ), srcq, dstl)

        out_specs = [pl.BlockSpec((1, 1, D), lambda g: (g, 0, 0))]
        out_shape = [jax.ShapeDtypeStruct((HB, 1, D), jnp.float32)]
        if with_z:
            out_specs = [_full((B, D))] + out_specs
            out_shape = [jax.ShapeDtypeStruct((B, D), jnp.float32)] + out_shape
        outs = pl.pallas_call(
            functools.partial(_tc3_body, with_z),
            grid=(HB,),
            in_specs=[_DEG_SPEC, _SEG_SPEC, _SEG_SPEC, _full((H,)),
                      _full((B, D)), _full((H, D)), _full((H,)),
                      _full((128, H)), _full((128,)),
                      _full((256, 128)), _full((256,)),
                      _full((D, 256)), _full((D,))],
            out_specs=out_specs,
            out_shape=out_shape,
        )(degh, s2, hp2, conv2_b, x, enc_W, enc_b,
          imp_W1, imp_b1, imp_W2, imp_b2, imp_W3, imp_b3)
        return outs

    z_norm, xiA = half_chain(0, True)
    (xiB,) = half_chain(HB, False)
    x_imp = jnp.concatenate([xiA.reshape(HB, D), xiB.reshape(HB, D)], axis=0)
    return (z_norm, x_imp)


# speedup vs baseline: 1.1338x; 1.0012x over previous
import functools

import jax
import jax.numpy as jnp
from jax import lax
from jax.experimental import pallas as pl
from jax.experimental.pallas import tpu as pltpu
from jax.experimental.pallas import tpu_sc as plsc

B, N, E, D, H = 16, 2048, 32768, 128, 128
NC, NS = 2, 16
CH = 128
DEGW = 16

_mesh = plsc.VectorSubcoreMesh(core_axis_name="core", subcore_axis_name="subcore")

Q = 4
DQ = D // Q
PB = 8
KB = (E // CH) // PB
KD2 = (E // NC) // CH


def _fill_rows(ref, value):
    rows, width = ref.shape

    @pl.loop(0, rows)
    def _(r):
        for l in range(width // 16):
            ref[r, pl.ds(l * 16, 16)] = jnp.full((16,), value, jnp.float32)


_DEG_SCRATCH = [
    pltpu.VMEM((CH, DEGW), jnp.float32),
    pltpu.VMEM((KD2, CH), jnp.int32),
    pltpu.VMEM((CH, DEGW), jnp.float32),
    pltpu.VMEM((N // CH, CH), jnp.int32),
    pltpu.VMEM((N, DEGW), jnp.float32),
    pltpu.VMEM_SHARED((NS, N, DEGW), jnp.float32),
]


def _build_identity(ref):
    iota = lax.iota(jnp.int32, 16)

    @pl.loop(0, ref.shape[0])
    def _(k):
        for l in range(CH // 16):
            ref[k, pl.ds(l * 16, 16)] = k * CH + l * 16 + iota


def _sc_degree_body(dst_hbm, out_hbm, ones_v, idx_v, zero_v, id_v, stage_v,
                    acc):
    c = lax.axis_index("core")
    s = lax.axis_index("subcore")
    g = s
    half = c

    _fill_rows(ones_v, 1.0)
    _fill_rows(zero_v, 0.0)
    _build_identity(id_v)
    my = acc.at[s]

    @pl.loop(0, N // CH)
    def _(k):
        pltpu.sync_copy(zero_v, my.at[id_v.at[k]])

    pltpu.sync_copy(dst_hbm.at[g, half], idx_v)

    @pl.loop(0, KD2)
    def _(k):
        pltpu.sync_copy(ones_v, my.at[idx_v.at[k]], add=True)

    pltpu.sync_copy(my, stage_v)
    pltpu.sync_copy(stage_v, out_hbm.at[g, half])


_SEG_SCRATCH = [
    pltpu.VMEM((KB, CH), jnp.int32),
    pltpu.VMEM((KB, CH), jnp.int32),
    pltpu.VMEM((6, CH, DQ), jnp.float32),
    pltpu.VMEM((CH, DQ), jnp.float32),
    pltpu.VMEM((N // CH, CH), jnp.int32),
    pltpu.SemaphoreType.DMA,
    pltpu.SemaphoreType.DMA,
    pltpu.VMEM_SHARED((NS, N, DQ), jnp.float32),
]


HB = B // NC


def _sc_segsum_body(hp_hbm, src_hbm, dst_hbm, out_hbm, src_v, dst_v,
                    rows_v, zero_v, id_v, gsem, ssem, acc):
    c = lax.axis_index("core")
    s = lax.axis_index("subcore")
    w = s * NC + c
    q = w // HB
    g = w % HB

    _fill_rows(zero_v, 0.0)
    _build_identity(id_v)
    my = acc.at[s]

    @pl.loop(0, N // CH)
    def _(k):
        pltpu.sync_copy(zero_v, my.at[id_v.at[k]])

    @pl.loop(0, PB)
    def _(p):
        pltpu.sync_copy(src_hbm.at[q, g, p], src_v)
        pltpu.sync_copy(dst_hbm.at[g, p], dst_v)

        gds = [None] * 6
        for j in range(min(3, KB)):
            gds[j] = pltpu.async_copy(hp_hbm.at[src_v.at[j]],
                                      rows_v.at[j], gsem)
        sds = [None, None]
        for k in range(KB):
            b = k % 6
            gds[b].wait()
            if sds[k & 1] is not None:
                sds[k & 1].wait()
            if k + 3 < KB:
                b2 = (k + 3) % 6
                gds[b2] = pltpu.async_copy(hp_hbm.at[src_v.at[k + 3]],
                                           rows_v.at[b2], gsem)
            sds[k & 1] = pltpu.async_copy(rows_v.at[b], my.at[dst_v.at[k]],
                                          ssem, add=True)
        for sd in sds:
            if sd is not None:
                sd.wait()

    pltpu.sync_copy(my, out_hbm.at[q, g])


_SC_PARAMS = pltpu.CompilerParams(use_tc_tiling_on_sc=False)

_sc_degree = pl.kernel(
    _sc_degree_body,
    out_type=jax.ShapeDtypeStruct((B, NC, N, DEGW), jnp.float32),
    mesh=_mesh,
    scratch_types=_DEG_SCRATCH,
    compiler_params=_SC_PARAMS,
)

_sc_segsum = pl.kernel(
    _sc_segsum_body,
    out_type=jax.ShapeDtypeStruct((Q, HB, N, DQ), jnp.float32),
    mesh=_mesh,
    scratch_types=_SEG_SCRATCH,
    compiler_params=_SC_PARAMS,
)


def _dinv_col(deg_ref):
    p = deg_ref[0, 0] + deg_ref[0, 1]
    return lax.rsqrt(p[:, 0:1] + 1.0)


def _cat(ref):
    return jnp.concatenate([ref[q, 0] for q in range(Q)], axis=1)


def _split_out(o_ref, mat):
    for q in range(Q):
        o_ref[q, 0] = mat[:, q * DQ:(q + 1) * DQ]


def _nt(a, b):
    return lax.dot_general(a, b, (((1,), (1,)), ((), ())),
                           preferred_element_type=jnp.float32)


def _tc1_body(deg_ref, x_ref, w_ref, hp_ref):
    dinv = _dinv_col(deg_ref)
    _split_out(hp_ref, _nt(x_ref[0], w_ref[...]) * dinv)


def _tc2_body(deg_ref, s_ref, hp_ref, b_ref, w_ref, o_ref):
    dinv = _dinv_col(deg_ref)
    gmat = jnp.maximum((_cat(s_ref) + _cat(hp_ref)) * dinv
                       + b_ref[...][None, :], 0.0)
    _split_out(o_ref, _nt(gmat, w_ref[...]) * dinv)


def _tc3_body(with_z, deg_ref, s_ref, hp_ref, b_ref, x_ref, encw_ref,
              encb_ref, w1_ref, b1_ref, w2_ref, b2_ref, w3_ref, b3_ref,
              *out_refs):
    if with_z:
        zn_ref, xi_ref = out_refs
    else:
        (xi_ref,) = out_refs
    g = pl.program_id(0)
    dinv = _dinv_col(deg_ref)
    out2 = (_cat(s_ref) + _cat(hp_ref)) * dinv + b_ref[...][None, :]
    emb = jnp.sum(out2, axis=0, keepdims=True) * (1.0 / N)
    embn = emb / jnp.maximum(jnp.sqrt(jnp.sum(emb * emb)), 1e-12)
    h1 = jnp.maximum(_nt(embn, w1_ref[...]) + b1_ref[...][None, :], 0.0)
    h2 = jnp.maximum(_nt(h1, w2_ref[...]) + b2_ref[...][None, :], 0.0)
    xi_ref[0] = _nt(h2, w3_ref[...]) + b3_ref[...][None, :]

    if with_z:
        @pl.when(g == 0)
        def _():
            z = _nt(x_ref[...], encw_ref[...]) + encb_ref[...][None, :]
            zn_ref[...] = z / jnp.maximum(
                jnp.sqrt(jnp.sum(z * z, axis=1, keepdims=True)), 1e-12)


def _full(shape):
    return pl.BlockSpec(shape, lambda g: tuple(0 for _ in shape))


_DEG_SPEC = pl.BlockSpec((1, NC, N, DEGW), lambda g: (g, 0, 0, 0))
_MAT_SPEC = pl.BlockSpec((1, N, D), lambda g: (g, 0, 0))
_SEG_SPEC = pl.BlockSpec((Q, 1, N, DQ), lambda g: (0, g, 0, 0))


def kernel(x, labels, loc, sub_x, sub_edge_index, enc_W, enc_b, conv1_W,
           conv1_b, conv2_W, conv2_b, imp_W1, imp_b1, imp_W2, imp_b2,
           imp_W3, imp_b3):
    src = sub_edge_index[:, 0, :].astype(jnp.int32)
    dst = sub_edge_index[:, 1, :].astype(jnp.int32)

    dst_deg = dst.reshape(B, NC, KD2, CH)
    degp = _sc_degree(dst_deg)

    qg_off = ((jnp.arange(Q, dtype=jnp.int32) * HB * N)[:, None]
              + (jnp.arange(HB, dtype=jnp.int32) * N)[None, :])

    def half_chain(lo, with_z):
        srch = src[lo:lo + HB]
        srcq = (srch[None] + qg_off[:, :, None]).reshape(Q, HB, PB, KB, CH)
        dstl = dst[lo:lo + HB].reshape(HB, PB, KB, CH)
        degh = degp[lo:lo + HB]

        hp1 = pl.pallas_call(
            _tc1_body,
            grid=(HB,),
            in_specs=[_DEG_SPEC, _MAT_SPEC, _full((H, D))],
            out_specs=_SEG_SPEC,
            out_shape=jax.ShapeDtypeStruct((Q, HB, N, DQ), jnp.float32),
        )(degh, sub_x[lo:lo + HB], conv1_W)

        s1 = _sc_segsum(hp1.reshape(Q * HB * N, DQ), srcq, dstl)

        hp2 = pl.pallas_call(
            _tc2_body,
            grid=(HB,),
            in_specs=[_DEG_SPEC, _SEG_SPEC, _SEG_SPEC, _full((H,)),
                      _full((H, H))],
            out_specs=_SEG_SPEC,
            out_shape=jax.ShapeDtypeStruct((Q, HB, N, DQ), jnp.float32),
        )(degh, s1, hp1, conv1_b, conv2_W)

        s2 = _sc_segsum(hp2.reshape(Q * HB * N, DQ), srcq, dstl)

        out_specs = [pl.BlockSpec((1, 1, D), lambda g: (g, 0, 0))]
        out_shape = [jax.ShapeDtypeStruct((HB, 1, D), jnp.float32)]
        if with_z:
            out_specs = [_full((B, D))] + out_specs
            out_shape = [jax.ShapeDtypeStruct((B, D), jnp.float32)] + out_shape
        outs = pl.pallas_call(
            functools.partial(_tc3_body, with_z),
            grid=(HB,),
            in_specs=[_DEG_SPEC, _SEG_SPEC, _SEG_SPEC, _full((H,)),
                      _full((B, D)), _full((H, D)), _full((H,)),
                      _full((128, H)), _full((128,)),
                      _full((256, 128)), _full((256,)),
                      _full((D, 256)), _full((D,))],
            out_specs=out_specs,
            out_shape=out_shape,
        )(degh, s2, hp2, conv2_b, x, enc_W, enc_b,
          imp_W1, imp_b1, imp_W2, imp_b2, imp_W3, imp_b3)
        return outs

    z_norm, xiA = half_chain(0, True)
    (xiB,) = half_chain(HB, False)
    x_imp = jnp.concatenate([xiA.reshape(HB, D), xiB.reshape(HB, D)], axis=0)
    return (z_norm, x_imp)
